# Initial kernel scaffold; baseline (speedup 1.0000x reference)
import jax
import jax.numpy as jnp
from jax.experimental import pallas as pl


def kernel(node_feats, params, edge_index, etype, graph_ids):
    raise NotImplementedError("write your pallas kernel here")


# speedup vs baseline: 1.5441x; 1.5441x over previous
import functools

import jax
import jax.numpy as jnp
from jax import lax
from jax.experimental import pallas as pl
from jax.experimental.pallas import tpu as pltpu
from jax.experimental.pallas import tpu_sc as plsc

N = 10000
E = 160000
D = 256
R = 16
B = 256
H = 128
EPS = 1e-5

HALF = N // 2
NPHASE = 2
DH = D // NPHASE
SPROWS = 5008
STRIPE = 312
EDGES_PER_TILE = E // 16
CHUNK = 80
NCHUNKS = EDGES_PER_TILE // CHUNK
NVREG = CHUNK // 16


def _mm_body(x_ref, w_ref, o_ref):
    o_ref[...] = jnp.dot(x_ref[...], w_ref[0],
                         preferred_element_type=jnp.float32)


def _rel_transform(x, w_rel):
    rows = 2000
    return pl.pallas_call(
        _mm_body,
        grid=(N // rows, R),
        in_specs=[
            pl.BlockSpec((rows, D), lambda i, j: (i, 0)),
            pl.BlockSpec((1, D, D), lambda i, j: (j, 0, 0)),
        ],
        out_specs=pl.BlockSpec((rows, D), lambda i, j: (i, j)),
        out_shape=jax.ShapeDtypeStruct((N, R * D), jnp.float32),
    )(x, w_rel)


def _sc_aggregate(xr_half, src, etype, dst):
    mesh = plsc.VectorSubcoreMesh(core_axis_name="c", subcore_axis_name="s")

    @functools.partial(
        pl.kernel,
        mesh=mesh,
        out_type=jax.ShapeDtypeStruct((NPHASE, N, DH), jnp.float32),
        scratch_types=[
            pltpu.VMEM((EDGES_PER_TILE,), jnp.int32),
            pltpu.VMEM((EDGES_PER_TILE,), jnp.int32),
            pltpu.VMEM((EDGES_PER_TILE,), jnp.int32),
            pltpu.VMEM((NCHUNKS, CHUNK), jnp.int32),
            pltpu.VMEM((NCHUNKS, CHUNK), jnp.int32),
            pltpu.VMEM((CHUNK, DH), jnp.float32),
            pltpu.VMEM((16, DH), jnp.float32),
            pltpu.VMEM_SHARED((SPROWS, DH), jnp.float32),
            pltpu.SemaphoreType.DMA,
        ],
    )
    def k(xr_hbm, src_hbm, et_hbm, dst_hbm, out_hbm,
          src_v, et_v, dst_v, idx2d, ldst2d, rows_v, zero_v, agg_sh, sem):
        cid = lax.axis_index("c")
        sid = lax.axis_index("s")
        ebase = sid * EDGES_PER_TILE

        pltpu.sync_copy(src_hbm.at[pl.ds(ebase, EDGES_PER_TILE)], src_v)
        pltpu.sync_copy(et_hbm.at[pl.ds(ebase, EDGES_PER_TILE)], et_v)
        pltpu.sync_copy(dst_hbm.at[pl.ds(ebase, EDGES_PER_TILE)], dst_v)

        nz = DH // 16

        def zfill(i, _):
            zero_v[i // nz, pl.ds((i % nz) * 16, 16)] = jnp.zeros(
                (16,), jnp.float32)
            return _
        lax.fori_loop(0, 16 * nz, zfill, 0)

        row_lo = cid * HALF

        def precomp(i, _):
            s = src_v[pl.ds(i * 16, 16)]
            t = et_v[pl.ds(i * 16, 16)]
            d = dst_v[pl.ds(i * 16, 16)]
            gq = (s * R + t) * NPHASE
            ld = d - row_lo
            ok = (ld >= 0) & (ld < HALF)
            dump = HALF + jnp.bitwise_and(d, 7)
            ld = jnp.where(ok, ld, dump)
            idx2d[i // NVREG, pl.ds((i % NVREG) * 16, 16)] = gq
            ldst2d[i // NVREG, pl.ds((i % NVREG) * 16, 16)] = ld
            return _
        lax.fori_loop(0, EDGES_PER_TILE // 16, precomp, 0)

        out_lo = cid * HALF + sid * STRIPE

        def run_phase(q):
            def zcopy(i, _):
                pltpu.sync_copy(zero_v,
                                agg_sh.at[pl.ds(sid * STRIPE + i * 16, 16)])
                return _
            lax.fori_loop(0, STRIPE // 16, zcopy, 0)
            pltpu.sync_copy(
                zero_v.at[pl.ds(0, 8)],
                agg_sh.at[pl.ds(sid * STRIPE + 16 * (STRIPE // 16), 8)])

            @pl.when(sid == 15)
            def _():
                pltpu.sync_copy(zero_v, agg_sh.at[pl.ds(16 * STRIPE, 16)])
            plsc.subcore_barrier()

            def chunk_body(ch, _):
                pltpu.async_copy(
                    xr_hbm.at[idx2d.at[ch]], rows_v, sem).wait()
                pltpu.sync_copy(rows_v, agg_sh.at[ldst2d.at[ch]], add=True)
                return _
            lax.fori_loop(0, NCHUNKS, chunk_body, 0)
            plsc.subcore_barrier()

            pltpu.sync_copy(agg_sh.at[pl.ds(sid * STRIPE, STRIPE)],
                            out_hbm.at[q, pl.ds(out_lo, STRIPE)])

            @pl.when(sid == 15)
            def _():
                pltpu.sync_copy(
                    agg_sh.at[pl.ds(16 * STRIPE, HALF - 16 * STRIPE)],
                    out_hbm.at[q, pl.ds(cid * HALF + 16 * STRIPE,
                                        HALF - 16 * STRIPE)])
            plsc.subcore_barrier()

        run_phase(0)

        def bump(i, _):
            r = idx2d[i // NVREG, pl.ds((i % NVREG) * 16, 16)]
            idx2d[i // NVREG, pl.ds((i % NVREG) * 16, 16)] = r + 1
            return _
        for q in range(1, NPHASE):
            lax.fori_loop(0, EDGES_PER_TILE // 16, bump, 0)
            run_phase(q)

    return k(xr_half, src, etype, dst)


def _post_body(agg_ref, x_ref, gid_ref,
               b_rel, res_W, res_b, bn_g, bn_b,
               att_w_row, att_b,
               fc1_W, fc1_b, bn1_g, bn1_b,
               fc2_W, fc2_b, bn2_g, bn2_b,
               fc3_W, fc3_b, bn3_g, bn3_b,
               out_W, out_b, o_ref):
    x = x_ref[...]
    agg3d = agg_ref[...]
    agg = jnp.concatenate([agg3d[q] for q in range(agg3d.shape[0])], axis=1)
    h = jnp.maximum(agg + b_rel[...], 0.0)
    res = jnp.maximum(
        jnp.dot(x, res_W[...], preferred_element_type=jnp.float32)
        + res_b[...], 0.0)
    h = h + res
    m = jnp.mean(h, axis=0, keepdims=True)
    v = jnp.mean((h - m) * (h - m), axis=0, keepdims=True)
    h = (h - m) / jnp.sqrt(v + EPS) * bn_g[...] + bn_b[...]
    z = jnp.sum(h * att_w_row[...], axis=1, keepdims=True) + att_b[...]
    w = 1.0 / (1.0 + jnp.exp(-z))
    hw = h * w
    sel = (lax.broadcasted_iota(jnp.int32, (B, N), 0)
           == gid_ref[...]).astype(jnp.float32)
    g = jnp.dot(sel, hw, preferred_element_type=jnp.float32)

    def fc(t, Wk, bk, gk, btk):
        y = jnp.maximum(
            jnp.dot(t, Wk[...], preferred_element_type=jnp.float32)
            + bk[...], 0.0)
        mm = jnp.mean(y, axis=0, keepdims=True)
        vv = jnp.mean((y - mm) * (y - mm), axis=0, keepdims=True)
        return (y - mm) / jnp.sqrt(vv + EPS) * gk[...] + btk[...]

    h1 = fc(g, fc1_W, fc1_b, bn1_g, bn1_b)
    h2 = fc(h1, fc2_W, fc2_b, bn2_g, bn2_b)
    h3 = fc(h2, fc3_W, fc3_b, bn3_g, bn3_b)
    o_ref[...] = (jnp.dot(h3, out_W[...], preferred_element_type=jnp.float32)
                  + out_b[...])


def _postprocess(agg, x, gid2d, p):
    args = (
        agg, x, gid2d,
        p['b_rel'].reshape(1, D), p['res_W'], p['res_b'].reshape(1, D),
        p['bn_g'].reshape(1, D), p['bn_b'].reshape(1, D),
        p['att_W'].reshape(1, D), p['att_b'].reshape(1, 1),
        p['fc1_W'], p['fc1_b'].reshape(1, H),
        p['bn1_g'].reshape(1, H), p['bn1_b'].reshape(1, H),
        p['fc2_W'], p['fc2_b'].reshape(1, H),
        p['bn2_g'].reshape(1, H), p['bn2_b'].reshape(1, H),
        p['fc3_W'], p['fc3_b'].reshape(1, H),
        p['bn3_g'].reshape(1, H), p['bn3_b'].reshape(1, H),
        p['out_W'], p['out_b'].reshape(1, 1),
    )
    return pl.pallas_call(
        _post_body,
        out_shape=jax.ShapeDtypeStruct((B, 1), jnp.float32),
    )(*args)


def kernel(node_feats, params, edge_index, etype, graph_ids):
    xr = _rel_transform(node_feats, params['W_rel'])
    xr_half = xr.reshape(N * R * NPHASE, DH)
    agg3d = _sc_aggregate(xr_half, edge_index[0], etype, edge_index[1])
    gid2d = graph_ids.reshape(1, N)
    return _postprocess(agg3d, node_feats, gid2d, params)


# speedup vs baseline: 1.6672x; 1.0797x over previous
import functools

import jax
import jax.numpy as jnp
from jax import lax
from jax.experimental import pallas as pl
from jax.experimental.pallas import tpu as pltpu
from jax.experimental.pallas import tpu_sc as plsc

N = 10000
E = 160000
D = 256
R = 16
B = 256
H = 128
EPS = 1e-5

NPHASE = 2
DH = D // NPHASE
SPROWS = N + 8
STRIPE = 624
EDGES_PER_TILE = E // 32
EPT_PAD = 5008
CHUNK = 96
NVREG = CHUNK // 16
NCHUNKS = 53
FULLV = EDGES_PER_TILE // 16


def _mm_body(x_ref, w_ref, o_ref):
    o_ref[...] = jnp.dot(x_ref[...], w_ref[0],
                         preferred_element_type=jnp.float32)


def _rel_transform(x, w_rel):
    rows = 2000
    return pl.pallas_call(
        _mm_body,
        grid=(N // rows, R),
        in_specs=[
            pl.BlockSpec((rows, D), lambda i, j: (i, 0)),
            pl.BlockSpec((1, D, D), lambda i, j: (j, 0, 0)),
        ],
        out_specs=pl.BlockSpec((rows, D), lambda i, j: (i, j)),
        out_shape=jax.ShapeDtypeStruct((N, R * D), jnp.float32),
    )(x, w_rel)


def _sc_aggregate(xr_q, src, etype, dst):
    mesh = plsc.VectorSubcoreMesh(core_axis_name="c", subcore_axis_name="s")

    @functools.partial(
        pl.kernel,
        mesh=mesh,
        out_type=jax.ShapeDtypeStruct((2 * NPHASE * N, DH), jnp.float32),
        scratch_types=[
            pltpu.VMEM((EPT_PAD,), jnp.int32),
            pltpu.VMEM((NCHUNKS, CHUNK), jnp.int32),
            pltpu.VMEM((NCHUNKS, CHUNK), jnp.int32),
            pltpu.VMEM((2, CHUNK, DH), jnp.float32),
            pltpu.VMEM((16, DH), jnp.float32),
            pltpu.VMEM_SHARED((SPROWS, DH), jnp.float32),
            pltpu.SemaphoreType.DMA,
            pltpu.SemaphoreType.DMA,
        ],
    )
    def k(xr_hbm, src_hbm, et_hbm, dst_hbm, out_hbm,
          slab, idx2d, ldst2d, rows2, zero_v, agg_sh, sem0, sem1):
        cid = lax.axis_index("c")
        sid = lax.axis_index("s")
        ebase = (cid * 16 + sid) * EDGES_PER_TILE

        def stage(hbm):
            pltpu.sync_copy(hbm.at[pl.ds(ebase, EDGES_PER_TILE)],
                            slab.at[pl.ds(0, EDGES_PER_TILE)])

        nz = DH // 16

        def zfill(i, c):
            zero_v[i // nz, pl.ds((i % nz) * 16, 16)] = jnp.zeros(
                (16,), jnp.float32)
            return c
        lax.fori_loop(0, 16 * nz, zfill, 0)

        lane = lax.broadcasted_iota(jnp.int32, (16,), 0)
        dumpv = N + jnp.bitwise_and(lane, 7)
        valid = lane < (EDGES_PER_TILE - FULLV * 16)
        tail_c = FULLV // NVREG
        tail_o = (FULLV % NVREG) * 16

        for v in range(NVREG):
            idx2d[NCHUNKS - 1, pl.ds(v * 16, 16)] = jnp.zeros((16,), jnp.int32)
            ldst2d[NCHUNKS - 1, pl.ds(v * 16, 16)] = dumpv

        stage(src_hbm)

        def p_src(i, c):
            idx2d[i // NVREG, pl.ds((i % NVREG) * 16, 16)] = (
                slab[pl.ds(i * 16, 16)] * (R * NPHASE))
            return c
        lax.fori_loop(0, FULLV, p_src, 0)
        sv = slab[pl.ds(FULLV * 16, 16)]
        idx2d[tail_c, pl.ds(tail_o, 16)] = jnp.where(
            valid, sv * (R * NPHASE), 0)

        stage(et_hbm)

        def p_et(i, c):
            cur = idx2d[i // NVREG, pl.ds((i % NVREG) * 16, 16)]
            idx2d[i // NVREG, pl.ds((i % NVREG) * 16, 16)] = (
                cur + slab[pl.ds(i * 16, 16)] * NPHASE)
            return c
        lax.fori_loop(0, FULLV, p_et, 0)
        tv = slab[pl.ds(FULLV * 16, 16)]
        cur = idx2d[tail_c, pl.ds(tail_o, 16)]
        idx2d[tail_c, pl.ds(tail_o, 16)] = cur + jnp.where(
            valid, tv * NPHASE, 0)

        stage(dst_hbm)

        def p_dst(i, c):
            ldst2d[i // NVREG, pl.ds((i % NVREG) * 16, 16)] = slab[
                pl.ds(i * 16, 16)]
            return c
        lax.fori_loop(0, FULLV, p_dst, 0)
        dv = slab[pl.ds(FULLV * 16, 16)]
        ldst2d[tail_c, pl.ds(tail_o, 16)] = jnp.where(valid, dv, dumpv)

        base = sid * STRIPE

        def run_phase(q):
            def zcopy(i, c):
                pltpu.sync_copy(zero_v, agg_sh.at[pl.ds(base + i * 16, 16)])
                return c
            lax.fori_loop(0, STRIPE // 16, zcopy, 0)

            @pl.when(sid == 15)
            def _():
                pltpu.sync_copy(zero_v, agg_sh.at[pl.ds(16 * STRIPE, 16)])
                pltpu.sync_copy(zero_v.at[pl.ds(0, 8)], agg_sh.at[pl.ds(N, 8)])
            plsc.subcore_barrier()

            pltpu.async_copy(xr_hbm.at[idx2d.at[0]], rows2.at[0], sem0)

            def chunk_body(ch, carry):
                nxt = ch + 1

                @pl.when((nxt < NCHUNKS) & (nxt % 2 == 0))
                def _():
                    pltpu.async_copy(xr_hbm.at[idx2d.at[nxt]], rows2.at[0],
                                     sem0)

                @pl.when((nxt < NCHUNKS) & (nxt % 2 == 1))
                def _():
                    pltpu.async_copy(xr_hbm.at[idx2d.at[nxt]], rows2.at[1],
                                     sem1)

                @pl.when(ch % 2 == 0)
                def _():
                    pltpu.make_async_copy(xr_hbm.at[pl.ds(0, CHUNK)],
                                          rows2.at[0], sem0).wait()
                    pltpu.sync_copy(rows2.at[0], agg_sh.at[ldst2d.at[ch]],
                                    add=True)

                @pl.when(ch % 2 == 1)
                def _():
                    pltpu.make_async_copy(xr_hbm.at[pl.ds(0, CHUNK)],
                                          rows2.at[1], sem1).wait()
                    pltpu.sync_copy(rows2.at[1], agg_sh.at[ldst2d.at[ch]],
                                    add=True)
                return carry
            lax.fori_loop(0, NCHUNKS, chunk_body, 0)
            plsc.subcore_barrier()

            obase = (cid * NPHASE + q) * N
            pltpu.sync_copy(agg_sh.at[pl.ds(base, STRIPE)],
                            out_hbm.at[pl.ds(obase + base, STRIPE)])

            @pl.when(sid == 15)
            def _():
                pltpu.sync_copy(
                    agg_sh.at[pl.ds(16 * STRIPE, N - 16 * STRIPE)],
                    out_hbm.at[pl.ds(obase + 16 * STRIPE, N - 16 * STRIPE)])
            plsc.subcore_barrier()

        run_phase(0)

        def bump(i, c):
            r = idx2d[i // NVREG, pl.ds((i % NVREG) * 16, 16)]
            idx2d[i // NVREG, pl.ds((i % NVREG) * 16, 16)] = r + 1
            return c
        for q in range(1, NPHASE):
            lax.fori_loop(0, NCHUNKS * NVREG, bump, 0)
            run_phase(q)

    return k(xr_q, src, etype, dst)


def _combine_body(a_ref, o_ref):
    a = a_ref[...]
    o_ref[...] = jnp.concatenate(
        [a[0, q] + a[1, q] for q in range(NPHASE)], axis=1)


def _combine(partials):
    rows = 2000
    return pl.pallas_call(
        _combine_body,
        grid=(N // rows,),
        in_specs=[pl.BlockSpec((2, NPHASE, rows, DH), lambda i: (0, 0, i, 0))],
        out_specs=pl.BlockSpec((rows, D), lambda i: (i, 0)),
        out_shape=jax.ShapeDtypeStruct((N, D), jnp.float32),
    )(partials)


def _post_body(agg_ref, x_ref, gid_ref,
               b_rel, res_W, res_b, bn_g, bn_b,
               att_w_row, att_b,
               fc1_W, fc1_b, bn1_g, bn1_b,
               fc2_W, fc2_b, bn2_g, bn2_b,
               fc3_W, fc3_b, bn3_g, bn3_b,
               out_W, out_b, o_ref):
    x = x_ref[...]
    agg = agg_ref[...]
    h = jnp.maximum(agg + b_rel[...], 0.0)
    res = jnp.maximum(
        jnp.dot(x, res_W[...], preferred_element_type=jnp.float32)
        + res_b[...], 0.0)
    h = h + res
    m = jnp.mean(h, axis=0, keepdims=True)
    v = jnp.mean((h - m) * (h - m), axis=0, keepdims=True)
    h = (h - m) / jnp.sqrt(v + EPS) * bn_g[...] + bn_b[...]
    z = jnp.sum(h * att_w_row[...], axis=1, keepdims=True) + att_b[...]
    w = 1.0 / (1.0 + jnp.exp(-z))
    hw = h * w
    sel = (lax.broadcasted_iota(jnp.int32, (B, N), 0)
           == gid_ref[...]).astype(jnp.float32)
    g = jnp.dot(sel, hw, preferred_element_type=jnp.float32)

    def fc(t, Wk, bk, gk, btk):
        y = jnp.maximum(
            jnp.dot(t, Wk[...], preferred_element_type=jnp.float32)
            + bk[...], 0.0)
        mm = jnp.mean(y, axis=0, keepdims=True)
        vv = jnp.mean((y - mm) * (y - mm), axis=0, keepdims=True)
        return (y - mm) / jnp.sqrt(vv + EPS) * gk[...] + btk[...]

    h1 = fc(g, fc1_W, fc1_b, bn1_g, bn1_b)
    h2 = fc(h1, fc2_W, fc2_b, bn2_g, bn2_b)
    h3 = fc(h2, fc3_W, fc3_b, bn3_g, bn3_b)
    o_ref[...] = (jnp.dot(h3, out_W[...], preferred_element_type=jnp.float32)
                  + out_b[...])


def _postprocess(agg, x, gid2d, p):
    args = (
        agg, x, gid2d,
        p['b_rel'].reshape(1, D), p['res_W'], p['res_b'].reshape(1, D),
        p['bn_g'].reshape(1, D), p['bn_b'].reshape(1, D),
        p['att_W'].reshape(1, D), p['att_b'].reshape(1, 1),
        p['fc1_W'], p['fc1_b'].reshape(1, H),
        p['bn1_g'].reshape(1, H), p['bn1_b'].reshape(1, H),
        p['fc2_W'], p['fc2_b'].reshape(1, H),
        p['bn2_g'].reshape(1, H), p['bn2_b'].reshape(1, H),
        p['fc3_W'], p['fc3_b'].reshape(1, H),
        p['bn3_g'].reshape(1, H), p['bn3_b'].reshape(1, H),
        p['out_W'], p['out_b'].reshape(1, 1),
    )
    return pl.pallas_call(
        _post_body,
        out_shape=jax.ShapeDtypeStruct((B, 1), jnp.float32),
    )(*args)


def kernel(node_feats, params, edge_index, etype, graph_ids):
    xr = _rel_transform(node_feats, params['W_rel'])
    xr_q = xr.reshape(N * R * NPHASE, DH)
    partials = _sc_aggregate(xr_q, edge_index[0], etype, edge_index[1])
    agg = _combine(partials.reshape(2, NPHASE, N, DH))
    gid2d = graph_ids.reshape(1, N)
    return _postprocess(agg, node_feats, gid2d, params)
